# baseline (speedup 1.0000x reference)
import functools

import jax
import jax.numpy as jnp
from jax import lax
from jax.experimental import pallas as pl
from jax.experimental.pallas import tpu as pltpu
from jax.experimental.pallas import tpu_sc as plsc

B = 16384
MF_D = 32
MLP_D = 64
H3 = 16

NC = 2
NS = 16
NW = NC * NS
BPW = B // NW
CH = 128
NCH = BPW // CH


def _sc_gather(user_ids, item_ids, mf_u_t, mf_i_t, ml_u_t, ml_i_t):
    mesh = plsc.VectorSubcoreMesh(core_axis_name="c", subcore_axis_name="s")

    @functools.partial(
        pl.kernel,
        mesh=mesh,
        compiler_params=pltpu.CompilerParams(use_tc_tiling_on_sc=False),
        out_type=[
            jax.ShapeDtypeStruct((B, MF_D), jnp.float32),
            jax.ShapeDtypeStruct((B, MF_D), jnp.float32),
            jax.ShapeDtypeStruct((B, MLP_D), jnp.float32),
            jax.ShapeDtypeStruct((B, MLP_D), jnp.float32),
        ],
        scratch_types=[
            pltpu.VMEM((NCH, CH), jnp.int32),
            pltpu.VMEM((NCH, CH), jnp.int32),
            pltpu.VMEM((BPW, MF_D), jnp.float32),
            pltpu.VMEM((BPW, MF_D), jnp.float32),
            pltpu.VMEM((BPW, MLP_D), jnp.float32),
            pltpu.VMEM((BPW, MLP_D), jnp.float32),
            pltpu.SemaphoreType.DMA,
        ],
    )
    def gather_kernel(uid_hbm, iid_hbm, mfu_t, mfi_t, mlu_t, mli_t,
                      mfu_o, mfi_o, mlu_o, mli_o,
                      uidx, iidx, mfu_v, mfi_v, mlu_v, mli_v, sem):
        wid = lax.axis_index("s") * NC + lax.axis_index("c")
        base = wid * BPW
        for c in range(NCH):
            pltpu.sync_copy(uid_hbm.at[pl.ds(base + c * CH, CH)], uidx.at[c])
            pltpu.sync_copy(iid_hbm.at[pl.ds(base + c * CH, CH)], iidx.at[c])
        handles = []
        for c in range(NCH):
            sl = pl.ds(c * CH, CH)
            handles.append(pltpu.async_copy(mfu_t.at[uidx.at[c]], mfu_v.at[sl], sem))
            handles.append(pltpu.async_copy(mfi_t.at[iidx.at[c]], mfi_v.at[sl], sem))
            handles.append(pltpu.async_copy(mlu_t.at[uidx.at[c]], mlu_v.at[sl], sem))
            handles.append(pltpu.async_copy(mli_t.at[iidx.at[c]], mli_v.at[sl], sem))
        for h in handles:
            h.wait()
        out_sl = pl.ds(base, BPW)
        pltpu.sync_copy(mfu_v, mfu_o.at[out_sl])
        pltpu.sync_copy(mfi_v, mfi_o.at[out_sl])
        pltpu.sync_copy(mlu_v, mlu_o.at[out_sl])
        pltpu.sync_copy(mli_v, mli_o.at[out_sl])

    return gather_kernel(user_ids, item_ids, mf_u_t, mf_i_t, ml_u_t, ml_i_t)


BS = 1024


def _tc_body(mfu_r, mfi_r, mlu_r, mli_r, w1u_r, w1i_r, b1_r, w2_r, b2_r,
             w3_r, b3_r, wmf_r, wh_r, bo_r, out_r):
    h = mlu_r[:] @ w1u_r[:] + mli_r[:] @ w1i_r[:] + b1_r[:]
    h = jnp.maximum(h, 0.0)
    h = jnp.maximum(h @ w2_r[:] + b2_r[:], 0.0)
    h = jnp.maximum(h @ w3_r[:] + b3_r[:], 0.0)
    mf = mfu_r[:] * mfi_r[:]
    s = jnp.sum(mf * wmf_r[:], axis=1) + jnp.sum(h * wh_r[:], axis=1) + bo_r[0, 0]
    out_r[:] = s


def _tc_mlp(mfu, mfi, mlu, mli, w1u, w1i, b1, w2, b2, w3, b3, wmf, wh, bo):
    grid = B // BS

    def batch_spec(d):
        return pl.BlockSpec((BS, d), lambda i: (i, 0))

    def full_spec(a, b):
        return pl.BlockSpec((a, b), lambda i: (0, 0))

    return pl.pallas_call(
        _tc_body,
        grid=(grid,),
        in_specs=[
            batch_spec(MF_D), batch_spec(MF_D),
            batch_spec(MLP_D), batch_spec(MLP_D),
            full_spec(MLP_D, MLP_D), full_spec(MLP_D, MLP_D), full_spec(1, MLP_D),
            full_spec(MLP_D, 32), full_spec(1, 32),
            full_spec(32, H3), full_spec(1, H3),
            full_spec(1, MF_D), full_spec(1, H3), full_spec(1, 1),
        ],
        out_specs=pl.BlockSpec((BS,), lambda i: (i,)),
        out_shape=jax.ShapeDtypeStruct((B,), jnp.float32),
    )(mfu, mfi, mlu, mli, w1u, w1i, b1, w2, b2, w3, b3, wmf, wh, bo)


def kernel(user_ids, item_ids, mf_user_table, mf_item_table, mlp_user_table,
           mlp_item_table, W1, b1, W2, b2, W3, b3, W_out, b_out):
    mfu, mfi, mlu, mli = _sc_gather(
        user_ids, item_ids, mf_user_table, mf_item_table,
        mlp_user_table, mlp_item_table)
    w1u = W1[:MLP_D]
    w1i = W1[MLP_D:]
    wmf = W_out[:MF_D, 0].reshape(1, MF_D)
    wh = W_out[MF_D:, 0].reshape(1, H3)
    return _tc_mlp(mfu, mfi, mlu, mli, w1u, w1i, b1.reshape(1, MLP_D),
                   W2, b2.reshape(1, 32), W3, b3.reshape(1, H3),
                   wmf, wh, b_out.reshape(1, 1))
